# Initial kernel scaffold; baseline (speedup 1.0000x reference)
import jax
import jax.numpy as jnp
from jax.experimental import pallas as pl


def kernel(mp_idx, mp_val, qx):
    raise NotImplementedError("write your pallas kernel here")


# speedup vs baseline: 7.5316x; 7.5316x over previous
import functools

import jax
import jax.numpy as jnp
from jax import lax
from jax.experimental import pallas as pl
from jax.experimental.pallas import tpu as pltpu
from jax.experimental.pallas import tpu_sc as plsc

_MAX_YR_LEN = 106
_T = _MAX_YR_LEN * 12
_B = 16384
_NC, _NS = 2, 16
_NW = _NC * _NS
_ROWS_PER_W = _B // _NW
_CHUNK = 64
_NCHUNK = _ROWS_PER_W // _CHUNK


@functools.partial(
    pl.kernel,
    out_type=jax.ShapeDtypeStruct((_B, _T), jnp.float32),
    mesh=plsc.VectorSubcoreMesh(core_axis_name="c", subcore_axis_name="s"),
    compiler_params=pltpu.CompilerParams(use_tc_tiling_on_sc=False),
    scratch_types=[
        pltpu.VMEM_SHARED((4, _T), jnp.float32),
        pltpu.VMEM((_ROWS_PER_W,), jnp.int32),
        pltpu.VMEM((_CHUNK, _T), jnp.float32),
        pltpu.SemaphoreType.DMA,
    ],
)
def _sc_lookup(table_hbm, idx_hbm, out_hbm, table_sh, idx_v, buf_v, sem):
    sid = lax.axis_index("s")
    wid = sid * _NC + lax.axis_index("c")
    base = wid * _ROWS_PER_W

    @pl.when(sid == 0)
    def _():
        pltpu.sync_copy(table_hbm, table_sh)

    pltpu.sync_copy(idx_hbm.at[pl.ds(base, _ROWS_PER_W)], idx_v)
    plsc.subcore_barrier()

    for k in range(_NCHUNK):
        pltpu.async_copy(
            table_sh.at[idx_v.at[pl.ds(k * _CHUNK, _CHUNK)]],
            buf_v, sem).wait()
        pltpu.sync_copy(buf_v, out_hbm.at[pl.ds(base + k * _CHUNK, _CHUNK)])


def kernel(mp_idx, mp_val, qx):
    del mp_val
    qm = jnp.power(qx + 1.0, 1.0 / 12.0) - 1.0
    rep = jnp.repeat(qm, 12, axis=1)
    sh = jnp.concatenate(
        [rep[:, 12:], jnp.zeros((2, 12), rep.dtype)], axis=1)
    table = jnp.stack([rep[0], sh[0], rep[1], sh[1]], axis=0)
    idx = (mp_idx[:, 0].astype(jnp.int32) * 2
           + mp_idx[:, 1].astype(jnp.int32))
    return _sc_lookup(table, idx)


# speedup vs baseline: 14.1920x; 1.8843x over previous
import functools

import jax
import jax.numpy as jnp
from jax import lax
from jax.experimental import pallas as pl
from jax.experimental.pallas import tpu as pltpu
from jax.experimental.pallas import tpu_sc as plsc

_MAX_YR_LEN = 106
_T = _MAX_YR_LEN * 12
_B = 16384
_NC, _NS = 2, 16
_NW = _NC * _NS
_ROWS_PER_W = _B // _NW
_L = 16
_GROUPS = _ROWS_PER_W // _L
_RING = 8


@functools.partial(
    pl.kernel,
    out_type=jax.ShapeDtypeStruct((_B, _T), jnp.float32),
    mesh=plsc.VectorSubcoreMesh(core_axis_name="c", subcore_axis_name="s"),
    compiler_params=pltpu.CompilerParams(needs_layout_passes=False),
    scratch_types=[
        pltpu.VMEM((4, _T), jnp.float32),
        pltpu.VMEM((_ROWS_PER_W,), jnp.int32),
        pltpu.SemaphoreType.DMA,
    ],
)
def _sc_lookup(table_hbm, idx_hbm, out_hbm, table_v, idx_v, sem):
    wid = lax.axis_index("s") * _NC + lax.axis_index("c")
    base = wid * _ROWS_PER_W
    pltpu.sync_copy(table_hbm, table_v)
    pltpu.sync_copy(idx_hbm.at[pl.ds(base, _ROWS_PER_W)], idx_v)

    lanes = lax.iota(jnp.int32, _L)

    def _fire(g, cvec, j):
        cj = lax.reduce_max(
            lax.select(lanes == j, cvec, jnp.zeros_like(cvec)), axes=(0,))
        pltpu.make_async_copy(
            table_v.at[cj], out_hbm.at[base + g * _L + j], sem).start()

    def _drain_one():
        pltpu.make_async_copy(table_v.at[0], out_hbm.at[base], sem).wait()

    cvec0 = idx_v[pl.ds(0, _L)]
    for j in range(_RING):
        _fire(0, cvec0, j)
    for j in range(_RING, _L):
        _drain_one()
        _fire(0, cvec0, j)

    @pl.loop(1, _GROUPS)
    def _body(g):
        cvec = idx_v[pl.ds(g * _L, _L)]
        for j in range(_L):
            _drain_one()
            _fire(g, cvec, j)

    for _ in range(_RING):
        _drain_one()


def kernel(mp_idx, mp_val, qx):
    del mp_val
    qm = jnp.power(qx + 1.0, 1.0 / 12.0) - 1.0
    rep = jnp.repeat(qm, 12, axis=1)
    sh = jnp.concatenate(
        [rep[:, 12:], jnp.zeros((2, 12), rep.dtype)], axis=1)
    table = jnp.stack([rep[0], sh[0], rep[1], sh[1]], axis=0)
    idx = (mp_idx[:, 0].astype(jnp.int32) * 2
           + mp_idx[:, 1].astype(jnp.int32))
    return _sc_lookup(table, idx)
